# baseline (speedup 1.0000x reference)
import functools

import jax
import jax.numpy as jnp
from jax import lax
from jax.experimental import pallas as pl
import jax.experimental.pallas.tpu as pltpu
from jax.experimental.pallas import tpu_sc as plsc

N_W = 100000
N_GRAPHS = 512
D_IN = 128
D_OUT = 32
BLK = 1000
NSTR = 4
N_BLK = N_W // BLK
N_STEP = N_BLK // NSTR
W = 64
NWIN = N_GRAPHS // W

NC = 2
NS = 16
NW_T = NC * NS
P = 100
SUB = 1000
NSTREAM = SUB // P
NCHUNK = N_W // SUB
KMAX = -(-NCHUNK // NW_T)


def _sums_fc(x_workload, batch3, seg_iota, cnts, fc_W, fc_b):
    def accum_block(x, seg, iota, acc_ref):
        xr = jnp.maximum(x, 0.0).astype(jnp.bfloat16)
        smin = jnp.min(seg)
        smax = jnp.max(seg)
        for t in range(NWIN):
            base = t * W

            @pl.when(jnp.logical_and(smin < base + W, smax >= base))
            def _win():
                oh = (iota + base == seg).astype(jnp.bfloat16)
                acc_ref[base:base + W, :] += jax.lax.dot_general(
                    oh, xr, (((1,), (0,)), ((), ())),
                    preferred_element_type=jnp.float32)

    def body(x0_ref, x1_ref, x2_ref, x3_ref, b0_ref, b1_ref, b2_ref, b3_ref,
             iota_ref, cnt_ref, w_ref, bias_ref, out_ref, acc_ref):
        i = pl.program_id(0)

        @pl.when(i == 0)
        def _init():
            acc_ref[...] = jnp.zeros_like(acc_ref)

        iota = iota_ref[...]
        accum_block(x0_ref[...], b0_ref[0], iota, acc_ref)
        accum_block(x1_ref[...], b1_ref[0], iota, acc_ref)
        accum_block(x2_ref[...], b2_ref[0], iota, acc_ref)
        accum_block(x3_ref[...], b3_ref[0], iota, acc_ref)

        @pl.when(i == N_STEP - 1)
        def _done():
            cnt = cnt_ref[0, :, 0:1] + cnt_ref[1, :, 0:1]
            emb = acc_ref[...] / jnp.maximum(cnt, 1.0)
            out_ref[...] = jax.lax.dot_general(
                emb, w_ref[...], (((1,), (0,)), ((), ())),
                preferred_element_type=jnp.float32) + bias_ref[...]

    def xspec(j):
        return pl.BlockSpec((BLK, D_IN), lambda i, j=j: (NSTR * i + j, 0))

    def bspec(j):
        return pl.BlockSpec((1, 1, BLK), lambda i, j=j: (NSTR * i + j, 0, 0))

    return pl.pallas_call(
        body,
        grid=(N_STEP,),
        in_specs=[
            xspec(0), xspec(1), xspec(2), xspec(3),
            bspec(0), bspec(1), bspec(2), bspec(3),
            pl.BlockSpec((W, BLK), lambda i: (0, 0)),
            pl.BlockSpec((NC, N_GRAPHS, 16), lambda i: (0, 0, 0)),
            pl.BlockSpec((D_IN, D_OUT), lambda i: (0, 0)),
            pl.BlockSpec((1, D_OUT), lambda i: (0, 0)),
        ],
        out_specs=pl.BlockSpec((N_GRAPHS, D_OUT), lambda i: (0, 0)),
        out_shape=jax.ShapeDtypeStruct((N_GRAPHS, D_OUT), jnp.float32),
        scratch_shapes=[
            pltpu.VMEM((N_GRAPHS, D_IN), jnp.float32),
        ],
    )(x_workload, x_workload, x_workload, x_workload,
      batch3, batch3, batch3, batch3, seg_iota, cnts, fc_W,
      fc_b.reshape(1, D_OUT))


def _sc_counts(batchc, zeros_cnt, ones_rows):
    mesh = plsc.VectorSubcoreMesh(core_axis_name="c", subcore_axis_name="s")

    @functools.partial(
        pl.kernel,
        mesh=mesh,
        compiler_params=pltpu.CompilerParams(use_tc_tiling_on_sc=False),
        out_type=jax.ShapeDtypeStruct((NC, N_GRAPHS, 16), jnp.float32),
        scratch_types=[
            pltpu.VMEM((NSTREAM, P), jnp.int32),
            pltpu.VMEM((P, 16), jnp.float32),
            pltpu.VMEM_SHARED((N_GRAPHS, 16), jnp.float32),
            pltpu.SemaphoreType.DMA,
        ],
    )
    def body(b_hbm, zcnt_hbm, ones_hbm, cnts_hbm, idx_v, ones_v, cnt_sh, sem):
        c = lax.axis_index("c")
        s = lax.axis_index("s")
        wid = c * NS + s

        @pl.when(s == 0)
        def _init():
            pltpu.sync_copy(zcnt_hbm, cnt_sh)

        pltpu.sync_copy(ones_hbm, ones_v)
        plsc.subcore_barrier()

        for k in range(KMAX):
            g = wid + k * NW_T

            @pl.when(g < NCHUNK)
            def _chunk():
                pltpu.sync_copy(b_hbm.at[g], idx_v)
                copies = []
                for j in range(NSTREAM):
                    copies.append(pltpu.async_copy(
                        ones_v, cnt_sh.at[idx_v.at[j]], sem, add=True))
                for cp in copies:
                    cp.wait()

        plsc.subcore_barrier()

        @pl.when(s == 0)
        def _flush():
            pltpu.sync_copy(cnt_sh, cnts_hbm.at[c])

    return body(batchc, zeros_cnt, ones_rows)


@jax.jit
def _pool_fc(x_workload, workload_batch, fc_W, fc_b):
    batch3 = workload_batch.reshape(N_BLK, 1, BLK)
    batchc = workload_batch.reshape(NCHUNK, NSTREAM, P)
    seg_iota = jax.lax.broadcasted_iota(jnp.int32, (W, BLK), 0)
    zeros_cnt = jnp.zeros((N_GRAPHS, 16), jnp.float32)
    ones_rows = jnp.ones((P, 16), jnp.float32)
    cnts = _sc_counts(batchc, zeros_cnt, ones_rows)
    return _sums_fc(x_workload, batch3, seg_iota, cnts, fc_W, fc_b)


def kernel(x_workload, x_vm, x_host, edge_index_assigned, edge_index_runs,
           workload_batch, conv1_gcn_W, conv1_gcn_b, conv1_sage_Wl,
           conv1_sage_Wr, conv1_sage_b, conv2_gcn_W, conv2_gcn_b,
           conv2_sage_Wl, conv2_sage_Wr, conv2_sage_b, fc_W, fc_b):
    return _pool_fc(x_workload, workload_batch, fc_W, fc_b)


# speedup vs baseline: 1.0864x; 1.0864x over previous
import functools

import jax
import jax.numpy as jnp
from jax import lax
from jax.experimental import pallas as pl
import jax.experimental.pallas.tpu as pltpu
from jax.experimental.pallas import tpu_sc as plsc

N_W = 100000
N_GRAPHS = 512
D_IN = 128
D_OUT = 32
BLK = 1000
NSTR = 4
N_BLK = N_W // BLK
N_STEP = N_BLK // NSTR
W = 64
NWIN = N_GRAPHS // W

NC = 2
NS = 16
NW_T = NC * NS
P = 125
SUB = 2000
NSTREAM = SUB // P
NCHUNK = N_W // SUB
KMAX = -(-NCHUNK // NW_T)


def _sums(x_workload, batch3, seg_iota):
    def accum_block(x, seg, iota, acc_ref):
        xr = jnp.maximum(x, 0.0).astype(jnp.bfloat16)
        smin = jnp.min(seg)
        smax = jnp.max(seg)
        for t in range(NWIN):
            base = t * W

            @pl.when(jnp.logical_and(smin < base + W, smax >= base))
            def _win():
                oh = (iota + base == seg).astype(jnp.bfloat16)
                acc_ref[base:base + W, :] += jax.lax.dot_general(
                    oh, xr, (((1,), (0,)), ((), ())),
                    preferred_element_type=jnp.float32)

    def body(x0_ref, x1_ref, x2_ref, x3_ref, b0_ref, b1_ref, b2_ref, b3_ref,
             iota_ref, out_ref, acc_ref):
        i = pl.program_id(0)

        @pl.when(i == 0)
        def _init():
            acc_ref[...] = jnp.zeros_like(acc_ref)

        iota = iota_ref[...]
        accum_block(x0_ref[...], b0_ref[0], iota, acc_ref)
        accum_block(x1_ref[...], b1_ref[0], iota, acc_ref)
        accum_block(x2_ref[...], b2_ref[0], iota, acc_ref)
        accum_block(x3_ref[...], b3_ref[0], iota, acc_ref)

        @pl.when(i == N_STEP - 1)
        def _done():
            out_ref[...] = acc_ref[...]

    def xspec(j):
        return pl.BlockSpec((BLK, D_IN), lambda i, j=j: (NSTR * i + j, 0))

    def bspec(j):
        return pl.BlockSpec((1, 1, BLK), lambda i, j=j: (NSTR * i + j, 0, 0))

    return pl.pallas_call(
        body,
        grid=(N_STEP,),
        in_specs=[
            xspec(0), xspec(1), xspec(2), xspec(3),
            bspec(0), bspec(1), bspec(2), bspec(3),
            pl.BlockSpec((W, BLK), lambda i: (0, 0)),
        ],
        out_specs=pl.BlockSpec((N_GRAPHS, D_IN), lambda i: (0, 0)),
        out_shape=jax.ShapeDtypeStruct((N_GRAPHS, D_IN), jnp.float32),
        scratch_shapes=[
            pltpu.VMEM((N_GRAPHS, D_IN), jnp.float32),
        ],
    )(x_workload, x_workload, x_workload, x_workload,
      batch3, batch3, batch3, batch3, seg_iota)


def _sc_counts(batchc, zeros_cnt, ones_rows):
    mesh = plsc.VectorSubcoreMesh(core_axis_name="c", subcore_axis_name="s")

    @functools.partial(
        pl.kernel,
        mesh=mesh,
        compiler_params=pltpu.CompilerParams(use_tc_tiling_on_sc=False),
        out_type=jax.ShapeDtypeStruct((NC, N_GRAPHS, 16), jnp.float32),
        scratch_types=[
            pltpu.VMEM((NSTREAM, P), jnp.int32),
            pltpu.VMEM((P, 16), jnp.float32),
            pltpu.VMEM_SHARED((N_GRAPHS, 16), jnp.float32),
            pltpu.SemaphoreType.DMA,
        ],
    )
    def body(b_hbm, zcnt_hbm, ones_hbm, cnts_hbm, idx_v, ones_v, cnt_sh, sem):
        c = lax.axis_index("c")
        s = lax.axis_index("s")
        wid = c * NS + s

        @pl.when(s == 0)
        def _init():
            pltpu.sync_copy(zcnt_hbm, cnt_sh)

        pltpu.sync_copy(ones_hbm, ones_v)
        plsc.subcore_barrier()

        for k in range(KMAX):
            g = wid + k * NW_T

            @pl.when(g < NCHUNK)
            def _chunk():
                pltpu.sync_copy(b_hbm.at[g], idx_v)
                copies = []
                for j in range(NSTREAM):
                    copies.append(pltpu.async_copy(
                        ones_v, cnt_sh.at[idx_v.at[j]], sem, add=True))
                for cp in copies:
                    cp.wait()

        plsc.subcore_barrier()

        @pl.when(s == 0)
        def _flush():
            pltpu.sync_copy(cnt_sh, cnts_hbm.at[c])

    return body(batchc, zeros_cnt, ones_rows)


def _finish_body(s_ref, c_ref, w_ref, bias_ref, out_ref):
    cnt = c_ref[0, :, 0:1] + c_ref[1, :, 0:1]
    emb = s_ref[...] / jnp.maximum(cnt, 1.0)
    out_ref[...] = jax.lax.dot_general(
        emb, w_ref[...], (((1,), (0,)), ((), ())),
        preferred_element_type=jnp.float32) + bias_ref[...]


def _finish(sums, cnts, fc_W, fc_b):
    return pl.pallas_call(
        _finish_body,
        in_specs=[
            pl.BlockSpec((N_GRAPHS, D_IN), lambda: (0, 0)),
            pl.BlockSpec((NC, N_GRAPHS, 16), lambda: (0, 0, 0)),
            pl.BlockSpec((D_IN, D_OUT), lambda: (0, 0)),
            pl.BlockSpec((1, D_OUT), lambda: (0, 0)),
        ],
        out_specs=pl.BlockSpec((N_GRAPHS, D_OUT), lambda: (0, 0)),
        out_shape=jax.ShapeDtypeStruct((N_GRAPHS, D_OUT), jnp.float32),
    )(sums, cnts, fc_W, fc_b.reshape(1, D_OUT))


@jax.jit
def _pool_fc(x_workload, workload_batch, fc_W, fc_b):
    batch3 = workload_batch.reshape(N_BLK, 1, BLK)
    batchc = workload_batch.reshape(NCHUNK, NSTREAM, P)
    seg_iota = jax.lax.broadcasted_iota(jnp.int32, (W, BLK), 0)
    zeros_cnt = jnp.zeros((N_GRAPHS, 16), jnp.float32)
    ones_rows = jnp.ones((P, 16), jnp.float32)
    cnts = _sc_counts(batchc, zeros_cnt, ones_rows)
    sums = _sums(x_workload, batch3, seg_iota)
    return _finish(sums, cnts, fc_W, fc_b)


def kernel(x_workload, x_vm, x_host, edge_index_assigned, edge_index_runs,
           workload_batch, conv1_gcn_W, conv1_gcn_b, conv1_sage_Wl,
           conv1_sage_Wr, conv1_sage_b, conv2_gcn_W, conv2_gcn_b,
           conv2_sage_Wl, conv2_sage_Wr, conv2_sage_b, fc_W, fc_b):
    return _pool_fc(x_workload, workload_batch, fc_W, fc_b)
